# Initial kernel scaffold; baseline (speedup 1.0000x reference)
import jax
import jax.numpy as jnp
from jax.experimental import pallas as pl


def kernel(input, feature, sp, num):
    raise NotImplementedError("write your pallas kernel here")


# speedup vs baseline: 17.3767x; 17.3767x over previous
import functools
import math

import jax
import jax.numpy as jnp
from jax import lax
from jax.experimental import pallas as pl
from jax.experimental.pallas import tpu as pltpu
from jax.experimental.pallas import tpu_sc as plsc

B = 4
HW = 224 * 224
NSEG = 196
NPAD = 208
NW = 32
PIX_PER_W = B * HW // NW
VECS_PER_W = PIX_PER_W // 16
SQRT3 = math.sqrt(3.0)
SQRT96 = math.sqrt(96.0)


def _resize_matrix(in_n, out_n):
    ys = jnp.linspace(0.0, in_n - 1.0, out_n)
    y0 = jnp.floor(ys).astype(jnp.int32)
    y1 = jnp.clip(y0 + 1, 0, in_n - 1)
    wy = ys - y0.astype(ys.dtype)
    ar = jnp.arange(in_n)
    oh0 = (ar[None, :] == y0[:, None]).astype(jnp.float32)
    oh1 = (ar[None, :] == y1[:, None]).astype(jnp.float32)
    return oh0 * (1.0 - wy)[:, None] + oh1 * wy[:, None]


def _stage_a_body(in_ref, feat_ref, ry_ref, ryt_ref, o1_ref, o2_ref):
    o1_ref[...] = jnp.sum(in_ref[...], axis=1)
    ry = ry_ref[...]
    ryt = ryt_ref[...]
    for b in range(B):
        fb = jnp.sum(feat_ref[b], axis=0)
        t = lax.dot(ry, fb, precision=lax.Precision.HIGHEST,
                    preferred_element_type=jnp.float32)
        ob = lax.dot(t, ryt, precision=lax.Precision.HIGHEST,
                     preferred_element_type=jnp.float32)
        o2_ref[b] = ob


_stage_a = pl.pallas_call(
    _stage_a_body,
    out_shape=[
        jax.ShapeDtypeStruct((B, HW), jnp.float32),
        jax.ShapeDtypeStruct((B, 224, 224), jnp.float32),
    ],
)


_sc_mesh = plsc.VectorSubcoreMesh(core_axis_name="c", subcore_axis_name="s")


@functools.partial(
    pl.kernel,
    mesh=_sc_mesh,
    compiler_params=pltpu.CompilerParams(needs_layout_passes=False),
    out_type=(
        jax.ShapeDtypeStruct((NW, NPAD), jnp.float32),
        jax.ShapeDtypeStruct((NW, NPAD), jnp.float32),
        jax.ShapeDtypeStruct((NW, NPAD), jnp.float32),
    ),
    scratch_types=(
        pltpu.VMEM((PIX_PER_W,), jnp.int32),
        pltpu.VMEM((PIX_PER_W,), jnp.float32),
        pltpu.VMEM((PIX_PER_W,), jnp.float32),
        pltpu.VMEM((NPAD,), jnp.float32),
        pltpu.VMEM((NPAD,), jnp.float32),
        pltpu.VMEM((NPAD,), jnp.float32),
    ),
)
def _sc_segment_sums(seg_hbm, v1_hbm, v2_hbm, o1_hbm, o2_hbm, oc_hbm,
                     seg_v, v1_v, v2_v, a1, a2, ac):
    wid = lax.axis_index("s") * 2 + lax.axis_index("c")
    pltpu.sync_copy(seg_hbm.at[wid], seg_v)
    pltpu.sync_copy(v1_hbm.at[wid], v1_v)
    pltpu.sync_copy(v2_hbm.at[wid], v2_v)

    zeros16 = jnp.zeros((16,), jnp.float32)

    def zero_body(i, c):
        a1[pl.ds(i * 16, 16)] = zeros16
        a2[pl.ds(i * 16, 16)] = zeros16
        ac[pl.ds(i * 16, 16)] = zeros16
        return c

    lax.fori_loop(0, NPAD // 16, zero_body, 0)

    ones16 = jnp.ones((16,), jnp.float32)

    def body(i, c):
        sl = pl.ds(i * 16, 16)
        idx = seg_v[sl]
        plsc.addupdate_scatter(a1, [idx], v1_v[sl])
        plsc.addupdate_scatter(a2, [idx], v2_v[sl])
        plsc.addupdate_scatter(ac, [idx], ones16)
        return c

    lax.fori_loop(0, VECS_PER_W, body, 0)

    pltpu.sync_copy(a1, o1_hbm.at[wid])
    pltpu.sync_copy(a2, o2_hbm.at[wid])
    pltpu.sync_copy(ac, oc_hbm.at[wid])


def _stage_c_body(p1_ref, p2_ref, pc_ref, p1t_ref, p2t_ref, pct_ref,
                  num_ref, out_ref):
    s1 = jnp.sum(p1_ref[...], axis=1)
    s2 = jnp.sum(p2_ref[...], axis=1)
    sc = jnp.sum(pc_ref[...], axis=1)
    p1t = p1t_ref[...]
    p2t = p2t_ref[...]
    pct = pct_ref[...]

    num0 = num_ref[0]
    ri = lax.broadcasted_iota(jnp.int32, (NPAD, NPAD), 0)
    ci = lax.broadcasted_iota(jnp.int32, (NPAD, NPAD), 1)
    valid = (ri < num0) & (ci < num0)
    col_iota = lax.broadcasted_iota(jnp.int32, (NPAD, 1), 0)
    row_iota = lax.broadcasted_iota(jnp.int32, (1, NPAD), 1)

    total = jnp.zeros((), jnp.float32)
    for b in range(B):
        nb = num_ref[b]
        s1r = lax.slice(s1, (b, 0), (b + 1, NPAD))
        s2r = lax.slice(s2, (b, 0), (b + 1, NPAD))
        scr = lax.slice(sc, (b, 0), (b + 1, NPAD))
        s1c = jnp.sum(lax.slice(p1t, (0, 8 * b), (NPAD, 8 * b + 8)),
                      axis=1, keepdims=True)
        s2c = jnp.sum(lax.slice(p2t, (0, 8 * b), (NPAD, 8 * b + 8)),
                      axis=1, keepdims=True)
        scc = jnp.sum(lax.slice(pct, (0, 8 * b), (NPAD, 8 * b + 8)),
                      axis=1, keepdims=True)

        okr = (scr > 0) & (row_iota < nb)
        okc = (scc > 0) & (col_iota < nb)
        m1r = jnp.where(okr, s1r / (jnp.maximum(scr, 1.0) * 3.0), 0.0)
        m2r = jnp.where(okr, s2r / (jnp.maximum(scr, 1.0) * 96.0), 0.0)
        m1c = jnp.where(okc, s1c / (jnp.maximum(scc, 1.0) * 3.0), 0.0)
        m2c = jnp.where(okc, s2c / (jnp.maximum(scc, 1.0) * 96.0), 0.0)

        d1 = m1c - m1r
        d2 = m2c - m2r
        n1 = jnp.sqrt(jnp.maximum(3.0 * d1 * d1, 1e-24)) / SQRT3
        n2 = jnp.sqrt(jnp.maximum(96.0 * d2 * d2, 1e-24)) / SQRT96
        total = total + jnp.sum(jnp.where(valid, jnp.abs(n2 - n1), 0.0))

    count = (4 * num0 * num0).astype(jnp.float32)
    out_ref[...] = (total / count) * jnp.ones((1, 1), jnp.float32)


_stage_c = pl.pallas_call(
    _stage_c_body,
    in_specs=[pl.BlockSpec(memory_space=pltpu.VMEM)] * 6
             + [pl.BlockSpec(memory_space=pltpu.SMEM)],
    out_shape=jax.ShapeDtypeStruct((1, 1), jnp.float32),
)


def kernel(input, feature, sp, num):
    x = input.astype(jnp.float32).reshape(B, 3, HW)
    f = feature.astype(jnp.float32)
    ry = _resize_matrix(56, 224)
    xsum1, xsum2 = _stage_a(x, f, ry, ry.T)

    seg32 = sp.reshape(B, HW).astype(jnp.int32).reshape(NW, PIX_PER_W)
    v1 = xsum1.reshape(NW, PIX_PER_W)
    v2 = xsum2.reshape(NW, PIX_PER_W)
    p1, p2, pc = _sc_segment_sums(seg32, v1, v2)

    num_i = num.astype(jnp.int32)
    out = _stage_c(p1.reshape(B, 8, NPAD), p2.reshape(B, 8, NPAD),
                   pc.reshape(B, 8, NPAD), p1.T, p2.T, pc.T, num_i)
    return out[0, 0]


# speedup vs baseline: 32.5158x; 1.8712x over previous
import functools
import math

import numpy as np
import jax
import jax.numpy as jnp
from jax import lax
from jax.experimental import pallas as pl
from jax.experimental.pallas import tpu as pltpu
from jax.experimental.pallas import tpu_sc as plsc

B = 4
H = 224
NSEG = 196
NPAD = 208
NW = 32
WPI = 7
ROWS_PER_W = H // WPI
SQRT3 = math.sqrt(3.0)
SQRT96 = math.sqrt(96.0)


def _resize_matrix_np(in_n, out_n):
    ys = np.linspace(0.0, in_n - 1.0, out_n, dtype=np.float32)
    y0 = np.floor(ys).astype(np.int32)
    y1 = np.clip(y0 + 1, 0, in_n - 1)
    wy = (ys - y0.astype(np.float32)).astype(np.float32)
    m = np.zeros((out_n, in_n), dtype=np.float32)
    m[np.arange(out_n), y0] += 1.0 - wy
    m[np.arange(out_n), y1] += wy
    return m


_RY = _resize_matrix_np(56, 224)


def _stage_a_body(in_ref, feat_ref, ry_ref, ryt_ref, o1_ref, o2_ref):
    ry = ry_ref[...]
    ryt = ryt_ref[...]
    for b in range(B):
        o1_ref[b] = jnp.sum(in_ref[b], axis=0)
        fb = jnp.sum(feat_ref[b], axis=2)
        t = lax.dot(ry, fb, precision=lax.Precision.HIGHEST,
                    preferred_element_type=jnp.float32)
        o2_ref[b] = lax.dot(t, ryt, precision=lax.Precision.HIGHEST,
                            preferred_element_type=jnp.float32)


_stage_a = pl.pallas_call(
    _stage_a_body,
    out_shape=[
        jax.ShapeDtypeStruct((B, H, H), jnp.float32),
        jax.ShapeDtypeStruct((B, H, H), jnp.float32),
    ],
)


_sc_mesh = plsc.VectorSubcoreMesh(core_axis_name="c", subcore_axis_name="s")


@functools.partial(
    pl.kernel,
    mesh=_sc_mesh,
    compiler_params=pltpu.CompilerParams(needs_layout_passes=False),
    out_type=(
        jax.ShapeDtypeStruct((NW, NPAD), jnp.float32),
        jax.ShapeDtypeStruct((NW, NPAD), jnp.float32),
        jax.ShapeDtypeStruct((NW, NPAD), jnp.float32),
    ),
    scratch_types=(
        pltpu.VMEM((ROWS_PER_W, H), jnp.int32),
        pltpu.VMEM((ROWS_PER_W, H), jnp.float32),
        pltpu.VMEM((ROWS_PER_W, H), jnp.float32),
        pltpu.VMEM((NPAD,), jnp.float32),
        pltpu.VMEM((NPAD,), jnp.float32),
        pltpu.VMEM((NPAD,), jnp.float32),
    ),
)
def _sc_segment_sums(seg_hbm, v1_hbm, v2_hbm, o1_hbm, o2_hbm, oc_hbm,
                     seg_s, v1_s, v2_s, a1, a2, ac):
    wid = lax.axis_index("s") * 2 + lax.axis_index("c")

    @pl.when(wid < B * WPI)
    def _():
        b = wid // WPI
        r0 = (wid % WPI) * ROWS_PER_W
        pltpu.sync_copy(seg_hbm.at[b, 0, pl.ds(r0, ROWS_PER_W), :], seg_s)
        pltpu.sync_copy(v1_hbm.at[b, pl.ds(r0, ROWS_PER_W), :], v1_s)
        pltpu.sync_copy(v2_hbm.at[b, pl.ds(r0, ROWS_PER_W), :], v2_s)

        zeros16 = jnp.zeros((16,), jnp.float32)

        def zero_body(i, c):
            a1[pl.ds(i * 16, 16)] = zeros16
            a2[pl.ds(i * 16, 16)] = zeros16
            ac[pl.ds(i * 16, 16)] = zeros16
            return c

        lax.fori_loop(0, NPAD // 16, zero_body, 0)

        ones16 = jnp.ones((16,), jnp.float32)

        def row_body(r, c):
            for k in range(H // 16):
                sl = pl.ds(k * 16, 16)
                idx = seg_s[r, sl]
                plsc.addupdate_scatter(a1, [idx], v1_s[r, sl])
                plsc.addupdate_scatter(a2, [idx], v2_s[r, sl])
                plsc.addupdate_scatter(ac, [idx], ones16)
            return c

        lax.fori_loop(0, ROWS_PER_W, row_body, 0)

        pltpu.sync_copy(a1, o1_hbm.at[wid])
        pltpu.sync_copy(a2, o2_hbm.at[wid])
        pltpu.sync_copy(ac, oc_hbm.at[wid])


def _stage_c_body(p1_ref, p2_ref, pc_ref, num_ref, out_ref):
    num0 = num_ref[0]
    ri = lax.broadcasted_iota(jnp.int32, (NPAD, NPAD), 0)
    ci = lax.broadcasted_iota(jnp.int32, (NPAD, NPAD), 1)
    valid = (ri < num0) & (ci < num0)
    row_iota = lax.broadcasted_iota(jnp.int32, (1, NPAD), 1)

    m1_rows = []
    m2_rows = []
    for b in range(B):
        s1r = jnp.sum(lax.slice(p1_ref[...], (WPI * b, 0), (WPI * b + WPI, NPAD)),
                      axis=0, keepdims=True)
        s2r = jnp.sum(lax.slice(p2_ref[...], (WPI * b, 0), (WPI * b + WPI, NPAD)),
                      axis=0, keepdims=True)
        scr = jnp.sum(lax.slice(pc_ref[...], (WPI * b, 0), (WPI * b + WPI, NPAD)),
                      axis=0, keepdims=True)
        okr = (scr > 0) & (row_iota < num_ref[b])
        m1_rows.append(jnp.where(okr, s1r / (jnp.maximum(scr, 1.0) * 3.0), 0.0))
        m2_rows.append(jnp.where(okr, s2r / (jnp.maximum(scr, 1.0) * 96.0), 0.0))

    m1 = lax.concatenate(m1_rows, 0)
    m2 = lax.concatenate(m2_rows, 0)
    m1t = jnp.transpose(m1)
    m2t = jnp.transpose(m2)

    total = jnp.zeros((), jnp.float32)
    for b in range(B):
        m1r = lax.slice(m1, (b, 0), (b + 1, NPAD))
        m2r = lax.slice(m2, (b, 0), (b + 1, NPAD))
        m1c = lax.slice(m1t, (0, b), (NPAD, b + 1))
        m2c = lax.slice(m2t, (0, b), (NPAD, b + 1))
        d1 = m1c - m1r
        d2 = m2c - m2r
        n1 = jnp.sqrt(jnp.maximum(3.0 * d1 * d1, 1e-24)) / SQRT3
        n2 = jnp.sqrt(jnp.maximum(96.0 * d2 * d2, 1e-24)) / SQRT96
        total = total + jnp.sum(jnp.where(valid, jnp.abs(n2 - n1), 0.0))

    count = (4 * num0 * num0).astype(jnp.float32)
    out_ref[...] = (total / count) * jnp.ones((1, 1), jnp.float32)


_stage_c = pl.pallas_call(
    _stage_c_body,
    in_specs=[pl.BlockSpec(memory_space=pltpu.VMEM)] * 3
             + [pl.BlockSpec(memory_space=pltpu.SMEM)],
    out_shape=jax.ShapeDtypeStruct((1, 1), jnp.float32),
)


def kernel(input, feature, sp, num):
    ry = jnp.asarray(_RY)
    ryt = jnp.asarray(_RY.T)
    xsum1, xsum2 = _stage_a(input, jnp.transpose(feature, (0, 2, 3, 1)), ry, ryt)
    p1, p2, pc = _sc_segment_sums(sp.astype(jnp.int32), xsum1, xsum2)
    out = _stage_c(p1, p2, pc, num.astype(jnp.int32))
    return out[0, 0]


# speedup vs baseline: 36.6758x; 1.1279x over previous
import functools
import math

import numpy as np
import jax
import jax.numpy as jnp
from jax import lax
from jax.experimental import pallas as pl
from jax.experimental.pallas import tpu as pltpu
from jax.experimental.pallas import tpu_sc as plsc

B = 4
H = 224
NSEG = 196
NPAD = 208
NW = 32
WPI = 7
ROWS_PER_W = H // WPI
SQRT3 = math.sqrt(3.0)
SQRT96 = math.sqrt(96.0)


def _resize_matrix_np(in_n, out_n):
    ys = np.linspace(0.0, in_n - 1.0, out_n, dtype=np.float32)
    y0 = np.floor(ys).astype(np.int32)
    y1 = np.clip(y0 + 1, 0, in_n - 1)
    wy = (ys - y0.astype(np.float32)).astype(np.float32)
    m = np.zeros((out_n, in_n), dtype=np.float32)
    m[np.arange(out_n), y0] += 1.0 - wy
    m[np.arange(out_n), y1] += wy
    return m


_RY = _resize_matrix_np(56, 224)


def _stage_a_body(in_ref, feat_ref, ry_ref, ryt_ref, o1_ref, o2_ref):
    ry = ry_ref[...]
    ryt = ryt_ref[...]
    for b in range(B):
        o1_ref[b] = jnp.sum(in_ref[b], axis=0)
        fb = jnp.sum(feat_ref[b], axis=2)
        t = lax.dot(ry, fb, precision=lax.Precision.HIGHEST,
                    preferred_element_type=jnp.float32)
        o2_ref[b] = lax.dot(t, ryt, precision=lax.Precision.HIGHEST,
                            preferred_element_type=jnp.float32)


_stage_a = pl.pallas_call(
    _stage_a_body,
    out_shape=[
        jax.ShapeDtypeStruct((B, H, H), jnp.float32),
        jax.ShapeDtypeStruct((B, H, H), jnp.float32),
    ],
)


_sc_mesh = plsc.VectorSubcoreMesh(core_axis_name="c", subcore_axis_name="s")


@functools.partial(
    pl.kernel,
    mesh=_sc_mesh,
    compiler_params=pltpu.CompilerParams(needs_layout_passes=False),
    out_type=jax.ShapeDtypeStruct((NW, 3 * NPAD), jnp.float32),
    scratch_types=(
        pltpu.VMEM((ROWS_PER_W, H), jnp.int32),
        pltpu.VMEM((ROWS_PER_W, H), jnp.float32),
        pltpu.VMEM((ROWS_PER_W, H), jnp.float32),
        pltpu.VMEM((3 * NPAD,), jnp.float32),
        pltpu.SemaphoreType.DMA,
    ),
)
def _sc_segment_sums(seg_hbm, v1_hbm, v2_hbm, out_hbm,
                     seg_s, v1_s, v2_s, acc, sem):
    wid = lax.axis_index("s") * 2 + lax.axis_index("c")

    @pl.when(wid < B * WPI)
    def _():
        b = wid // WPI
        r0 = (wid % WPI) * ROWS_PER_W
        rs = pl.ds(r0, ROWS_PER_W)
        cps = [
            pltpu.async_copy(seg_hbm.at[b, 0, rs, :], seg_s, sem),
            pltpu.async_copy(v1_hbm.at[b, rs, :], v1_s, sem),
            pltpu.async_copy(v2_hbm.at[b, rs, :], v2_s, sem),
        ]

        zeros16 = jnp.zeros((16,), jnp.float32)

        def zero_body(i, c):
            acc[pl.ds(i * 16, 16)] = zeros16
            return c

        lax.fori_loop(0, 3 * NPAD // 16, zero_body, 0)
        for cp in cps:
            cp.wait()

        ones16 = jnp.ones((16,), jnp.float32)
        off1 = jnp.full((16,), NPAD, jnp.int32)
        off2 = jnp.full((16,), 2 * NPAD, jnp.int32)
        GRP = 7

        def row_body(r, c):
            for g in range(H // 16 // GRP):
                sls = [pl.ds((g * GRP + j) * 16, 16) for j in range(GRP)]
                idxs = [seg_s[r, sl] for sl in sls]
                v1l = [v1_s[r, sl] for sl in sls]
                v2l = [v2_s[r, sl] for sl in sls]
                for j in range(GRP):
                    plsc.addupdate_scatter(acc, [idxs[j]], v1l[j])
                    plsc.addupdate_scatter(acc, [idxs[j] + off1], v2l[j])
                    plsc.addupdate_scatter(acc, [idxs[j] + off2], ones16)
            return c

        lax.fori_loop(0, ROWS_PER_W, row_body, 0)

        pltpu.sync_copy(acc, out_hbm.at[wid])


def _stage_c_body(p_ref, num_ref, out_ref):
    pall = p_ref[...]
    num0 = num_ref[0]
    ri = lax.broadcasted_iota(jnp.int32, (NPAD, NPAD), 0)
    ci = lax.broadcasted_iota(jnp.int32, (NPAD, NPAD), 1)
    valid = (ri < num0) & (ci < num0)
    row_iota = lax.broadcasted_iota(jnp.int32, (1, NPAD), 1)

    m1_rows = []
    m2_rows = []
    for b in range(B):
        grp = jnp.sum(lax.slice(pall, (WPI * b, 0), (WPI * b + WPI, 3 * NPAD)),
                      axis=0, keepdims=True)
        s1r = lax.slice(grp, (0, 0), (1, NPAD))
        s2r = lax.slice(grp, (0, NPAD), (1, 2 * NPAD))
        scr = lax.slice(grp, (0, 2 * NPAD), (1, 3 * NPAD))
        okr = (scr > 0) & (row_iota < num_ref[b])
        m1_rows.append(jnp.where(okr, s1r / (jnp.maximum(scr, 1.0) * 3.0), 0.0))
        m2_rows.append(jnp.where(okr, s2r / (jnp.maximum(scr, 1.0) * 96.0), 0.0))

    m1 = lax.concatenate(m1_rows, 0)
    m2 = lax.concatenate(m2_rows, 0)
    m1t = jnp.transpose(m1)
    m2t = jnp.transpose(m2)

    total = jnp.zeros((), jnp.float32)
    for b in range(B):
        m1r = lax.slice(m1, (b, 0), (b + 1, NPAD))
        m2r = lax.slice(m2, (b, 0), (b + 1, NPAD))
        m1c = lax.slice(m1t, (0, b), (NPAD, b + 1))
        m2c = lax.slice(m2t, (0, b), (NPAD, b + 1))
        d1 = m1c - m1r
        d2 = m2c - m2r
        n1 = jnp.sqrt(jnp.maximum(3.0 * d1 * d1, 1e-24)) / SQRT3
        n2 = jnp.sqrt(jnp.maximum(96.0 * d2 * d2, 1e-24)) / SQRT96
        total = total + jnp.sum(jnp.where(valid, jnp.abs(n2 - n1), 0.0))

    count = (4 * num0 * num0).astype(jnp.float32)
    out_ref[...] = (total / count) * jnp.ones((1, 1), jnp.float32)


_stage_c = pl.pallas_call(
    _stage_c_body,
    in_specs=[pl.BlockSpec(memory_space=pltpu.VMEM),
              pl.BlockSpec(memory_space=pltpu.SMEM)],
    out_shape=jax.ShapeDtypeStruct((1, 1), jnp.float32),
)


def kernel(input, feature, sp, num):
    ry = jnp.asarray(_RY)
    ryt = jnp.asarray(_RY.T)
    xsum1, xsum2 = _stage_a(input, jnp.transpose(feature, (0, 2, 3, 1)), ry, ryt)
    partials = _sc_segment_sums(sp.astype(jnp.int32), xsum1, xsum2)
    out = _stage_c(partials, num.astype(jnp.int32))
    return out[0, 0]


# speedup vs baseline: 36.8247x; 1.0041x over previous
import functools
import math

import numpy as np
import jax
import jax.numpy as jnp
from jax import lax
from jax.experimental import pallas as pl
from jax.experimental.pallas import tpu as pltpu
from jax.experimental.pallas import tpu_sc as plsc

B = 4
H = 224
NSEG = 196
NPAD = 208
NW = 32
WPI = 8
PROC_ROWS = H // WPI
WIN_ROWS = 32
SQRT3 = math.sqrt(3.0)
SQRT96 = math.sqrt(96.0)


def _resize_matrix_np(in_n, out_n):
    ys = np.linspace(0.0, in_n - 1.0, out_n, dtype=np.float32)
    y0 = np.floor(ys).astype(np.int32)
    y1 = np.clip(y0 + 1, 0, in_n - 1)
    wy = (ys - y0.astype(np.float32)).astype(np.float32)
    m = np.zeros((out_n, in_n), dtype=np.float32)
    m[np.arange(out_n), y0] += 1.0 - wy
    m[np.arange(out_n), y1] += wy
    return m


_RY = _resize_matrix_np(56, 224)


def _stage_a_body(in_ref, feat_ref, ry_ref, ryt_ref, o1_ref, o2_ref):
    ry = ry_ref[...]
    ryt = ryt_ref[...]
    for b in range(B):
        o1_ref[b] = jnp.sum(in_ref[b], axis=0)
        fb = jnp.sum(feat_ref[b], axis=2)
        t = lax.dot(ry, fb, precision=lax.Precision.HIGHEST,
                    preferred_element_type=jnp.float32)
        o2_ref[b] = lax.dot(t, ryt, precision=lax.Precision.HIGHEST,
                            preferred_element_type=jnp.float32)


_stage_a = pl.pallas_call(
    _stage_a_body,
    out_shape=[
        jax.ShapeDtypeStruct((B, H, H), jnp.float32),
        jax.ShapeDtypeStruct((B, H, H), jnp.float32),
    ],
)


_sc_mesh = plsc.VectorSubcoreMesh(core_axis_name="c", subcore_axis_name="s")


@functools.partial(
    pl.kernel,
    mesh=_sc_mesh,
    compiler_params=pltpu.CompilerParams(needs_layout_passes=False),
    out_type=jax.ShapeDtypeStruct((NW, 3 * NPAD), jnp.float32),
    scratch_types=(
        pltpu.VMEM((WIN_ROWS, H), jnp.int32),
        pltpu.VMEM((WIN_ROWS, H), jnp.float32),
        pltpu.VMEM((WIN_ROWS, H), jnp.float32),
        pltpu.VMEM((3 * NPAD,), jnp.float32),
        pltpu.SemaphoreType.DMA,
    ),
)
def _sc_segment_sums(seg_hbm, v1_hbm, v2_hbm, out_hbm,
                     seg_s, v1_s, v2_s, acc, sem):
    wid = lax.axis_index("s") * 2 + lax.axis_index("c")
    b = wid // WPI
    chunk = wid % WPI
    d = 4 * (chunk % 2)
    r0 = pl.multiple_of(PROC_ROWS * chunk - d, 8)
    rs = pl.ds(r0, WIN_ROWS)
    cps = [
        pltpu.async_copy(seg_hbm.at[b, 0, rs, :], seg_s, sem),
        pltpu.async_copy(v1_hbm.at[b, rs, :], v1_s, sem),
        pltpu.async_copy(v2_hbm.at[b, rs, :], v2_s, sem),
    ]

    zeros16 = jnp.zeros((16,), jnp.float32)

    def zero_body(i, c):
        acc[pl.ds(i * 16, 16)] = zeros16
        return c

    lax.fori_loop(0, 3 * NPAD // 16, zero_body, 0)
    for cp in cps:
        cp.wait()

    ones16 = jnp.ones((16,), jnp.float32)
    off1 = jnp.full((16,), NPAD, jnp.int32)
    off2 = jnp.full((16,), 2 * NPAD, jnp.int32)
    GRP = 7

    def row_body(r, c):
        rr = r + d
        for g in range(H // 16 // GRP):
            sls = [pl.ds((g * GRP + j) * 16, 16) for j in range(GRP)]
            idxs = [seg_s[rr, sl] for sl in sls]
            v1l = [v1_s[rr, sl] for sl in sls]
            v2l = [v2_s[rr, sl] for sl in sls]
            for j in range(GRP):
                plsc.addupdate_scatter(acc, [idxs[j]], v1l[j])
                plsc.addupdate_scatter(acc, [idxs[j] + off1], v2l[j])
                plsc.addupdate_scatter(acc, [idxs[j] + off2], ones16)
        return c

    lax.fori_loop(0, PROC_ROWS, row_body, 0)

    pltpu.sync_copy(acc, out_hbm.at[wid])


def _stage_c_body(p_ref, num_ref, out_ref):
    pall = p_ref[...]
    num0 = num_ref[0]
    ri = lax.broadcasted_iota(jnp.int32, (NPAD, NPAD), 0)
    ci = lax.broadcasted_iota(jnp.int32, (NPAD, NPAD), 1)
    valid = (ri < num0) & (ci < num0)
    row_iota = lax.broadcasted_iota(jnp.int32, (1, NPAD), 1)

    m1_rows = []
    m2_rows = []
    for b in range(B):
        grp = jnp.sum(lax.slice(pall, (WPI * b, 0), (WPI * b + WPI, 3 * NPAD)),
                      axis=0, keepdims=True)
        s1r = lax.slice(grp, (0, 0), (1, NPAD))
        s2r = lax.slice(grp, (0, NPAD), (1, 2 * NPAD))
        scr = lax.slice(grp, (0, 2 * NPAD), (1, 3 * NPAD))
        okr = (scr > 0) & (row_iota < num_ref[b])
        m1_rows.append(jnp.where(okr, s1r / (jnp.maximum(scr, 1.0) * 3.0), 0.0))
        m2_rows.append(jnp.where(okr, s2r / (jnp.maximum(scr, 1.0) * 96.0), 0.0))

    m1 = lax.concatenate(m1_rows, 0)
    m2 = lax.concatenate(m2_rows, 0)
    m1t = jnp.transpose(m1)
    m2t = jnp.transpose(m2)

    total = jnp.zeros((), jnp.float32)
    for b in range(B):
        m1r = lax.slice(m1, (b, 0), (b + 1, NPAD))
        m2r = lax.slice(m2, (b, 0), (b + 1, NPAD))
        m1c = lax.slice(m1t, (0, b), (NPAD, b + 1))
        m2c = lax.slice(m2t, (0, b), (NPAD, b + 1))
        d1 = m1c - m1r
        d2 = m2c - m2r
        n1 = jnp.sqrt(jnp.maximum(3.0 * d1 * d1, 1e-24)) / SQRT3
        n2 = jnp.sqrt(jnp.maximum(96.0 * d2 * d2, 1e-24)) / SQRT96
        total = total + jnp.sum(jnp.where(valid, jnp.abs(n2 - n1), 0.0))

    count = (4 * num0 * num0).astype(jnp.float32)
    out_ref[...] = (total / count) * jnp.ones((1, 1), jnp.float32)


_stage_c = pl.pallas_call(
    _stage_c_body,
    in_specs=[pl.BlockSpec(memory_space=pltpu.VMEM),
              pl.BlockSpec(memory_space=pltpu.SMEM)],
    out_shape=jax.ShapeDtypeStruct((1, 1), jnp.float32),
)


def kernel(input, feature, sp, num):
    ry = jnp.asarray(_RY)
    ryt = jnp.asarray(_RY.T)
    xsum1, xsum2 = _stage_a(input, jnp.transpose(feature, (0, 2, 3, 1)), ry, ryt)
    partials = _sc_segment_sums(sp.astype(jnp.int32), xsum1, xsum2)
    out = _stage_c(partials, num.astype(jnp.int32))
    return out[0, 0]
